# baseline (speedup 1.0000x reference)
import functools

import jax
import jax.numpy as jnp
from jax import lax
from jax.experimental import pallas as pl
from jax.experimental.pallas import tpu as pltpu
from jax.experimental.pallas import tpu_sc as plsc


def _build(B, D):
    info = plsc.get_sparse_core_info()
    nw = info.num_cores * info.num_subcores
    b_per_w = B // nw
    assert B % (8 * nw) == 0

    mesh = plsc.VectorSubcoreMesh(core_axis_name="c", subcore_axis_name="s")

    @functools.partial(
        pl.kernel,
        mesh=mesh,
        out_type=jax.ShapeDtypeStruct((B, D), jnp.float32),
        scratch_types=[
            pltpu.VMEM((b_per_w,), jnp.int32),
            pltpu.VMEM((b_per_w,), jnp.int32),
            pltpu.VMEM((b_per_w, D), jnp.float32),
            pltpu.SemaphoreType.DMA,
        ],
        compiler_params=pltpu.CompilerParams(use_tc_tiling_on_sc=False),
    )
    def k(nid_hbm, map_hbm, table_hbm, out_hbm, nid_v, idx_v, rows_v, sem):
        wid = lax.axis_index("s") * info.num_cores + lax.axis_index("c")
        base = wid * b_per_w
        pltpu.sync_copy(nid_hbm.at[pl.ds(base, b_per_w)], nid_v)
        pltpu.async_copy(map_hbm.at[nid_v], idx_v, sem).wait()
        pltpu.async_copy(table_hbm.at[idx_v], rows_v, sem).wait()
        pltpu.sync_copy(rows_v, out_hbm.at[pl.ds(base, b_per_w)])

    return k


@jax.jit
def kernel(nid, nid_map, emb_table):
    B = nid.shape[0]
    D = emb_table.shape[1]
    k = _build(B, D)
    return k(nid, nid_map, emb_table)


# speedup vs baseline: 1.0314x; 1.0314x over previous
import functools

import jax
import jax.numpy as jnp
from jax import lax
from jax.experimental import pallas as pl
from jax.experimental.pallas import tpu as pltpu
from jax.experimental.pallas import tpu_sc as plsc


def _build(B, D, V1):
    info = plsc.get_sparse_core_info()
    nw = info.num_cores * info.num_subcores
    b_per_w = B // nw
    assert B % (8 * nw) == 0

    mesh = plsc.VectorSubcoreMesh(core_axis_name="c", subcore_axis_name="s")

    @functools.partial(
        pl.kernel,
        mesh=mesh,
        out_type=jax.ShapeDtypeStruct((B, D), jnp.float32),
        scratch_types=[
            pltpu.VMEM((b_per_w,), jnp.int32),
            pltpu.VMEM((b_per_w,), jnp.int32),
            pltpu.SemaphoreType.DMA,
            pltpu.SemaphoreType.DMA,
        ],
    )
    def k(nid_hbm, map_hbm, table_hbm, out_hbm, nid_v, idx_v, sem, rsem):
        wid = lax.axis_index("s") * info.num_cores + lax.axis_index("c")
        base = wid * b_per_w
        pltpu.sync_copy(nid_hbm.at[pl.ds(base, b_per_w)], nid_v)
        pltpu.async_copy(map_hbm.at[nid_v], idx_v, sem).wait()

        L = info.num_lanes

        def _fire(g, _):
            v = idx_v[pl.ds(g * L, L)]
            for j in range(L):
                pltpu.async_copy(
                    table_hbm.at[pl.ds(v[j], 1)],
                    out_hbm.at[pl.ds(base + g * L + j, 1)],
                    rsem,
                )
            return 0

        lax.fori_loop(0, b_per_w // L, _fire, 0)

        def _drain(i, _):
            pltpu.make_async_copy(
                table_hbm.at[pl.ds(0, 1)],
                out_hbm.at[pl.ds(base + i, 1)],
                rsem,
            ).wait()
            return 0

        lax.fori_loop(0, b_per_w, _drain, 0)

    return k


@jax.jit
def kernel(nid, nid_map, emb_table):
    B = nid.shape[0]
    V1, D = emb_table.shape
    k = _build(B, D, V1)
    return k(nid, nid_map, emb_table)


# speedup vs baseline: 1.7166x; 1.6644x over previous
import functools

import jax
import jax.numpy as jnp
from jax import lax
from jax.experimental import pallas as pl
from jax.experimental.pallas import tpu as pltpu
from jax.experimental.pallas import tpu_sc as plsc


def _build(B, D, V1):
    info = plsc.get_sparse_core_info()
    nw = info.num_cores * info.num_subcores
    b_per_w = B // nw
    mesh = plsc.VectorSubcoreMesh(core_axis_name="c", subcore_axis_name="s")

    @functools.partial(
        pl.kernel,
        mesh=mesh,
        out_type=jax.ShapeDtypeStruct((B, D), jnp.float32),
        scratch_types=[
            pltpu.VMEM((b_per_w,), jnp.int32),
            pltpu.VMEM((b_per_w,), jnp.int32),
            pltpu.VMEM((b_per_w, D), jnp.float32),
            pltpu.SemaphoreType.DMA,
            pltpu.SemaphoreType.DMA,
        ],
    )
    def k(nid_hbm, map_hbm, table_hbm, out_hbm, nid_v, idx_v, rows_v, sem, rsem):
        wid = lax.axis_index("s") * info.num_cores + lax.axis_index("c")
        base = wid * b_per_w
        pltpu.sync_copy(nid_hbm.at[pl.ds(base, b_per_w)], nid_v)
        pltpu.async_copy(map_hbm.at[nid_v], idx_v, sem).wait()

        L = info.num_lanes

        def _fire(g, _):
            v = idx_v[pl.ds(g * L, L)]
            for j in range(L):
                pltpu.async_copy(
                    table_hbm.at[pl.ds(v[j], 1)],
                    rows_v.at[pl.ds(g * L + j, 1)],
                    rsem,
                )
            return 0

        lax.fori_loop(0, b_per_w // L, _fire, 0)

        def _drain(i, _):
            pltpu.make_async_copy(
                table_hbm.at[pl.ds(0, 1)],
                rows_v.at[pl.ds(i, 1)],
                rsem,
            ).wait()
            return 0

        lax.fori_loop(0, b_per_w, _drain, 0)
        pltpu.sync_copy(rows_v, out_hbm.at[pl.ds(base, b_per_w)])

    return k


@jax.jit
def kernel(nid, nid_map, emb_table):
    B = nid.shape[0]
    V1, D = emb_table.shape
    k = _build(B, D, V1)
    return k(nid, nid_map, emb_table)
